# baseline (speedup 1.0000x reference)
import jax
import jax.numpy as jnp
from jax.experimental import pallas as pl
from jax.experimental.pallas import tpu as pltpu

_N = 200000
_ROWS = 8000
_GRID = _N // _ROWS


def _zero_fill(res_points_ref, res_coors_ref):
    res_points_ref[...] = jnp.zeros(res_points_ref.shape, jnp.float32)
    res_coors_ref[...] = jnp.zeros(res_coors_ref.shape, jnp.int32)


def kernel(points):
    del points
    res_points, res_coors = pl.pallas_call(
        _zero_fill,
        grid=(_GRID,),
        out_specs=[
            pl.BlockSpec((_ROWS, 6), lambda i: (i, 0)),
            pl.BlockSpec((_ROWS, 4), lambda i: (i, 0)),
        ],
        out_shape=[
            jax.ShapeDtypeStruct((_N, 6), jnp.float32),
            jax.ShapeDtypeStruct((_N, 4), jnp.int32),
        ],
        compiler_params=pltpu.CompilerParams(
            dimension_semantics=("arbitrary",),
        ),
    )()
    return (res_points, res_coors)
